# baseline (speedup 1.0000x reference)
import functools

import jax
import jax.numpy as jnp
from jax import lax
from jax.experimental import pallas as pl
from jax.experimental.pallas import tpu as pltpu
from jax.experimental.pallas import tpu_sc as plsc

N_NODES = 10000
N_EDGES = 320000
ND = 128
ED = 128
HID = 256

NC, NS = 2, 16
NW = NC * NS
N_PAD = 10240
ROWS_PER_TILE = N_PAD // NS
NBUF = 5

HALF = N_EDGES // 2
E_PER_W = HALF // NW
CHUNK = 40
N_CHUNKS = E_PER_W // CHUNK

_MESH = plsc.VectorSubcoreMesh(
    core_axis_name="c", subcore_axis_name="s", num_cores=NC, num_subcores=NS)

_EPS = 1e-5


def _make_gather(edge_base):
    @functools.partial(
        pl.kernel,
        out_type=[
            jax.ShapeDtypeStruct((HALF, ND), jnp.float32),
            jax.ShapeDtypeStruct((HALF, ND), jnp.float32),
        ],
        mesh=_MESH,
        scratch_types=[
            pltpu.VMEM((E_PER_W,), jnp.int32),
            pltpu.VMEM((E_PER_W,), jnp.int32),
            pltpu.VMEM((NBUF, CHUNK, ND), jnp.float32),
            pltpu.VMEM((NBUF, CHUNK, ND), jnp.float32),
            pltpu.SemaphoreType.DMA((NBUF,)),
            pltpu.SemaphoreType.DMA((NBUF,)),
            pltpu.SemaphoreType.DMA((NBUF,)),
            pltpu.SemaphoreType.DMA((NBUF,)),
        ],
        name=f"gather_h{edge_base // HALF}",
    )
    def gather(senders, receivers, nodes, s_out, r_out,
               sidx, ridx, sbuf, rbuf, sgsem, rgsem, swsem, rwsem):
        wid = lax.axis_index("s") * NC + lax.axis_index("c")
        base = wid * E_PER_W
        pltpu.sync_copy(senders.at[pl.ds(edge_base + base, E_PER_W)], sidx)
        pltpu.sync_copy(receivers.at[pl.ds(edge_base + base, E_PER_W)], ridx)

        def fire_gather(c, b):
            pltpu.async_copy(nodes.at[sidx.at[pl.ds(c * CHUNK, CHUNK)]],
                             sbuf.at[b], sgsem.at[b])
            pltpu.async_copy(nodes.at[ridx.at[pl.ds(c * CHUNK, CHUNK)]],
                             rbuf.at[b], rgsem.at[b])

        for b in range(NBUF):
            fire_gather(b, b)

        @pl.loop(NBUF, N_CHUNKS + NBUF, step=NBUF)
        def _(i):
            for b in range(NBUF):
                c = i - NBUF + b
                off = base + c * CHUNK
                pltpu.make_async_copy(
                    nodes.at[sidx.at[pl.ds(c * CHUNK, CHUNK)]],
                    sbuf.at[b], sgsem.at[b]).wait()
                pltpu.async_copy(sbuf.at[b], s_out.at[pl.ds(off, CHUNK)],
                                 swsem.at[b])
                pltpu.make_async_copy(
                    nodes.at[ridx.at[pl.ds(c * CHUNK, CHUNK)]],
                    rbuf.at[b], rgsem.at[b]).wait()
                pltpu.async_copy(rbuf.at[b], r_out.at[pl.ds(off, CHUNK)],
                                 rwsem.at[b])
            for b in range(NBUF):
                c = i - NBUF + b
                off = base + c * CHUNK
                pltpu.make_async_copy(sbuf.at[b],
                                      s_out.at[pl.ds(off, CHUNK)],
                                      swsem.at[b]).wait()
                pltpu.make_async_copy(rbuf.at[b],
                                      r_out.at[pl.ds(off, CHUNK)],
                                      rwsem.at[b]).wait()

                @pl.when(i + b < N_CHUNKS)
                def _():
                    fire_gather(i + b, b)

    return gather


def _make_scatter(edge_base):
    @functools.partial(
        pl.kernel,
        out_type=jax.ShapeDtypeStruct((NC, N_PAD, ED), jnp.float32),
        mesh=_MESH,
        scratch_types=[
            pltpu.VMEM((NBUF, CHUNK), jnp.int32),
            pltpu.VMEM((NBUF, CHUNK, ED), jnp.float32),
            pltpu.VMEM_SHARED((N_PAD, ED), jnp.float32),
            pltpu.SemaphoreType.DMA((NBUF,)),
            pltpu.SemaphoreType.DMA((NBUF,)),
        ],
        name=f"scatter_h{edge_base // HALF}",
    )
    def scatter(receivers, msgs, zeros, agg_out, ibuf, mbuf, acc, isem,
                msem):
        cid = lax.axis_index("c")
        sid = lax.axis_index("s")
        wid = sid * NC + cid
        stripe = sid * ROWS_PER_TILE
        pltpu.sync_copy(zeros.at[pl.ds(stripe, ROWS_PER_TILE)],
                        acc.at[pl.ds(stripe, ROWS_PER_TILE)])
        plsc.subcore_barrier()

        base = wid * E_PER_W

        def fire_load(c, b):
            off = base + c * CHUNK
            pltpu.async_copy(receivers.at[pl.ds(edge_base + off, CHUNK)],
                             ibuf.at[b], isem.at[b])
            pltpu.async_copy(msgs.at[pl.ds(off, CHUNK)], mbuf.at[b],
                             msem.at[b])

        for b in range(NBUF):
            fire_load(b, b)

        @pl.loop(NBUF, N_CHUNKS + NBUF, step=NBUF)
        def _(i):
            for b in range(NBUF):
                c = i - NBUF + b
                off = base + c * CHUNK
                pltpu.make_async_copy(
                    receivers.at[pl.ds(edge_base + off, CHUNK)],
                    ibuf.at[b], isem.at[b]).wait()
                pltpu.make_async_copy(msgs.at[pl.ds(off, CHUNK)],
                                      mbuf.at[b], msem.at[b]).wait()
                pltpu.sync_copy(mbuf.at[b], acc.at[ibuf.at[b]], add=True)

                @pl.when(i + b < N_CHUNKS)
                def _():
                    fire_load(i + b, b)

        plsc.subcore_barrier()
        pltpu.sync_copy(acc.at[pl.ds(stripe, ROWS_PER_TILE)],
                        agg_out.at[cid, pl.ds(stripe, ROWS_PER_TILE)])

    return scatter


_gather_h0 = _make_gather(0)
_gather_h1 = _make_gather(HALF)
_scatter_h0 = _make_scatter(0)
_scatter_h1 = _make_scatter(HALF)


def _edge_mlp_body(e_ref, s_ref, r_ref, we_e, we_s, we_r, be1, gg, gb,
                   we2, be2, out_ref):
    bf = jnp.bfloat16
    x = (jnp.dot(e_ref[...].astype(bf), we_e[...].astype(bf),
                 preferred_element_type=jnp.float32)
         + jnp.dot(s_ref[...].astype(bf), we_s[...].astype(bf),
                   preferred_element_type=jnp.float32)
         + jnp.dot(r_ref[...].astype(bf), we_r[...].astype(bf),
                   preferred_element_type=jnp.float32)
         + be1[...])
    m = jnp.mean(x, axis=-1, keepdims=True)
    v = jnp.mean((x - m) * (x - m), axis=-1, keepdims=True)
    h = (x - m) * lax.rsqrt(v + _EPS) * gg[...] + gb[...]
    h = jnp.maximum(h, 0.0)
    out_ref[...] = (jnp.dot(h.astype(bf), we2[...].astype(bf),
                            preferred_element_type=jnp.float32) + be2[...])


_EBLK = 2000


def _edge_mlp_half(edges_ac_full, s_feat, r_feat, we_e, we_s, we_r, be1,
                   gg, gb, we2, be2, half_idx):
    grid = (HALF // _EBLK,)
    off_blocks = half_idx * (HALF // _EBLK)
    eblk = lambda i: (i + off_blocks, 0)
    blk = lambda i: (i, 0)
    full = lambda i: (0, 0)
    return pl.pallas_call(
        _edge_mlp_body,
        grid=grid,
        in_specs=[
            pl.BlockSpec((_EBLK, ED), eblk),
            pl.BlockSpec((_EBLK, ND), blk),
            pl.BlockSpec((_EBLK, ND), blk),
            pl.BlockSpec((ED, HID), full),
            pl.BlockSpec((ND, HID), full),
            pl.BlockSpec((ND, HID), full),
            pl.BlockSpec((1, HID), full),
            pl.BlockSpec((1, HID), full),
            pl.BlockSpec((1, HID), full),
            pl.BlockSpec((HID, ED), full),
            pl.BlockSpec((1, ED), full),
        ],
        out_specs=pl.BlockSpec((_EBLK, ED), blk),
        out_shape=jax.ShapeDtypeStruct((HALF, ED), jnp.float32),
    )(edges_ac_full, s_feat, r_feat, we_e, we_s, we_r, be1, gg, gb, we2,
      be2)


def _node_mlp_body(n_ref, a0_ref, a1_ref, a2_ref, a3_ref, wn_n, wn_a, bn1,
                   gg, gb, wn2, bn2, out_ref):
    bf = jnp.bfloat16
    agg = (a0_ref[...] + a1_ref[...]) + (a2_ref[...] + a3_ref[...])
    x = (jnp.dot(n_ref[...].astype(bf), wn_n[...].astype(bf),
                 preferred_element_type=jnp.float32)
         + jnp.dot(agg.astype(bf), wn_a[...].astype(bf),
                   preferred_element_type=jnp.float32)
         + bn1[...])
    m = jnp.mean(x, axis=-1, keepdims=True)
    v = jnp.mean((x - m) * (x - m), axis=-1, keepdims=True)
    h = (x - m) * lax.rsqrt(v + _EPS) * gg[...] + gb[...]
    h = jnp.maximum(h, 0.0)
    out_ref[...] = (jnp.dot(h.astype(bf), wn2[...].astype(bf),
                            preferred_element_type=jnp.float32) + bn2[...])


_NBLK = 1000


def _node_mlp(nodes, a0, a1, a2, a3, wn_n, wn_a, bn1, gg, gb, wn2, bn2):
    grid = (N_NODES // _NBLK,)
    blk = lambda i: (i, 0)
    full = lambda i: (0, 0)
    return pl.pallas_call(
        _node_mlp_body,
        grid=grid,
        in_specs=[
            pl.BlockSpec((_NBLK, ND), blk),
            pl.BlockSpec((_NBLK, ED), blk),
            pl.BlockSpec((_NBLK, ED), blk),
            pl.BlockSpec((_NBLK, ED), blk),
            pl.BlockSpec((_NBLK, ED), blk),
            pl.BlockSpec((ND, HID), full),
            pl.BlockSpec((ED, HID), full),
            pl.BlockSpec((1, HID), full),
            pl.BlockSpec((1, HID), full),
            pl.BlockSpec((1, HID), full),
            pl.BlockSpec((HID, ND), full),
            pl.BlockSpec((1, ND), full),
        ],
        out_specs=pl.BlockSpec((_NBLK, ND), blk),
        out_shape=jax.ShapeDtypeStruct((N_NODES, ND), jnp.float32),
    )(nodes, a0, a1, a2, a3, wn_n, wn_a, bn1, gg, gb, wn2, bn2)


def kernel(nodes_bus, edges_ac, edge_index, We1, be1, ge_g, ge_b, We2, be2,
           Wn1, bn1, gn_g, gn_b, Wn2, bn2):
    senders = edge_index[0]
    receivers = edge_index[1]

    s0, r0 = _gather_h0(senders, receivers, nodes_bus)
    s1, r1 = _gather_h1(senders, receivers, nodes_bus)

    we_e, we_s, we_r = We1[:ED], We1[ED:ED + ND], We1[ED + ND:]
    be1r = be1.reshape(1, HID)
    ggr, gbr = ge_g.reshape(1, HID), ge_b.reshape(1, HID)
    be2r = be2.reshape(1, ED)

    ue0 = _edge_mlp_half(edges_ac, s0, r0, we_e, we_s, we_r,
                         be1r, ggr, gbr, We2, be2r, 0)
    ue1 = _edge_mlp_half(edges_ac, s1, r1, we_e, we_s, we_r,
                         be1r, ggr, gbr, We2, be2r, 1)

    zeros = jnp.zeros((N_PAD, ED), jnp.float32)
    agg0 = _scatter_h0(receivers, ue0, zeros)
    agg1 = _scatter_h1(receivers, ue1, zeros)

    updated_nodes = _node_mlp(
        nodes_bus, agg0[0], agg0[1], agg1[0], agg1[1],
        Wn1[:ND], Wn1[ND:],
        bn1.reshape(1, HID), gn_g.reshape(1, HID), gn_b.reshape(1, HID),
        Wn2, bn2.reshape(1, ND))

    updated_edges = jnp.concatenate([ue0, ue1], axis=0)
    return (updated_nodes, updated_edges)


# speedup vs baseline: 1.3297x; 1.3297x over previous
import functools

import jax
import jax.numpy as jnp
from jax import lax
from jax.experimental import pallas as pl
from jax.experimental.pallas import tpu as pltpu
from jax.experimental.pallas import tpu_sc as plsc

N_NODES = 10000
N_EDGES = 320000
ND = 128
ED = 128
HID = 256

NC, NS = 2, 16
NW = NC * NS
E_PER_W = N_EDGES // NW
CHUNK = 80
N_CHUNKS = E_PER_W // CHUNK
N_PAD = 10240
ROWS_PER_TILE = N_PAD // NS
CHUNK_SC = 40
N_CHUNKS_SC = E_PER_W // CHUNK_SC

_MESH = plsc.VectorSubcoreMesh(
    core_axis_name="c", subcore_axis_name="s", num_cores=NC, num_subcores=NS)

_EPS = 1e-5


NBUF = 5
CHUNK_G = 40
N_CHUNKS_G = E_PER_W // CHUNK_G
STAGE_ROWS = 640
LAST_STAGE = N_NODES - 15 * STAGE_ROWS


@functools.partial(
    pl.kernel,
    out_type=[
        jax.ShapeDtypeStruct((N_EDGES, ND), jnp.float32),
        jax.ShapeDtypeStruct((N_EDGES, ND), jnp.float32),
    ],
    mesh=_MESH,
    scratch_types=[
        pltpu.VMEM((E_PER_W,), jnp.int32),
        pltpu.VMEM((NBUF, CHUNK_G, ND), jnp.float32),
        pltpu.VMEM_SHARED((N_NODES, ND), jnp.float32),
        pltpu.SemaphoreType.DMA((NBUF,)),
        pltpu.SemaphoreType.DMA((NBUF,)),
    ],
)
def _gather_sc(senders, receivers, nodes, s_out, r_out,
               idx, buf, table, gsem, wsem):
    cid = lax.axis_index("c")
    sid = lax.axis_index("s")
    wid = sid * NC + cid
    base = wid * E_PER_W

    @pl.when(sid < NS - 1)
    def _():
        pltpu.sync_copy(nodes.at[pl.ds(sid * STAGE_ROWS, STAGE_ROWS)],
                        table.at[pl.ds(sid * STAGE_ROWS, STAGE_ROWS)])

    @pl.when(sid == NS - 1)
    def _():
        pltpu.sync_copy(nodes.at[pl.ds(15 * STAGE_ROWS, LAST_STAGE)],
                        table.at[pl.ds(15 * STAGE_ROWS, LAST_STAGE)])

    plsc.subcore_barrier()

    def one_pass(idx_hbm, out_hbm):
        pltpu.sync_copy(idx_hbm.at[pl.ds(base, E_PER_W)], idx)

        def fire_gather(c, b):
            pltpu.async_copy(table.at[idx.at[pl.ds(c * CHUNK_G, CHUNK_G)]],
                             buf.at[b], gsem.at[b])

        for b in range(NBUF):
            fire_gather(b, b)

        @pl.loop(NBUF, N_CHUNKS_G + NBUF, step=NBUF)
        def _(i):
            for b in range(NBUF):
                c = i - NBUF + b
                off = base + c * CHUNK_G
                pltpu.make_async_copy(
                    table.at[idx.at[pl.ds(c * CHUNK_G, CHUNK_G)]],
                    buf.at[b], gsem.at[b]).wait()
                pltpu.async_copy(buf.at[b], out_hbm.at[pl.ds(off, CHUNK_G)],
                                 wsem.at[b])
            for b in range(NBUF):
                c = i - NBUF + b
                off = base + c * CHUNK_G
                pltpu.make_async_copy(buf.at[b],
                                      out_hbm.at[pl.ds(off, CHUNK_G)],
                                      wsem.at[b]).wait()

                @pl.when(i + b < N_CHUNKS_G)
                def _():
                    fire_gather(i + b, b)

    one_pass(senders, s_out)
    one_pass(receivers, r_out)


@functools.partial(
    pl.kernel,
    out_type=jax.ShapeDtypeStruct((NC, N_PAD, ED), jnp.float32),
    mesh=_MESH,
    scratch_types=[
        pltpu.VMEM((NBUF, CHUNK_SC), jnp.int32),
        pltpu.VMEM((NBUF, CHUNK_SC, ED), jnp.float32),
        pltpu.VMEM_SHARED((N_PAD, ED), jnp.float32),
        pltpu.SemaphoreType.DMA((NBUF,)),
        pltpu.SemaphoreType.DMA((NBUF,)),
    ],
)
def _scatter_sc(receivers, msgs, zeros, agg_out, ibuf, mbuf, acc, isem, msem):
    cid = lax.axis_index("c")
    sid = lax.axis_index("s")
    wid = sid * NC + cid
    stripe = sid * ROWS_PER_TILE
    pltpu.sync_copy(zeros.at[pl.ds(stripe, ROWS_PER_TILE)],
                    acc.at[pl.ds(stripe, ROWS_PER_TILE)])
    plsc.subcore_barrier()

    base = wid * E_PER_W

    def fire_load(c, b):
        off = base + c * CHUNK_SC
        pltpu.async_copy(receivers.at[pl.ds(off, CHUNK_SC)], ibuf.at[b],
                         isem.at[b])
        pltpu.async_copy(msgs.at[pl.ds(off, CHUNK_SC)], mbuf.at[b],
                         msem.at[b])

    for b in range(NBUF):
        fire_load(b, b)

    @pl.loop(NBUF, N_CHUNKS_SC + NBUF, step=NBUF)
    def _(i):
        for b in range(NBUF):
            c = i - NBUF + b
            off = base + c * CHUNK_SC
            pltpu.make_async_copy(receivers.at[pl.ds(off, CHUNK_SC)],
                                  ibuf.at[b], isem.at[b]).wait()
            pltpu.make_async_copy(msgs.at[pl.ds(off, CHUNK_SC)],
                                  mbuf.at[b], msem.at[b]).wait()
            pltpu.sync_copy(mbuf.at[b], acc.at[ibuf.at[b]], add=True)

            @pl.when(i + b < N_CHUNKS_SC)
            def _():
                fire_load(i + b, b)

    plsc.subcore_barrier()
    pltpu.sync_copy(acc.at[pl.ds(stripe, ROWS_PER_TILE)],
                    agg_out.at[cid, pl.ds(stripe, ROWS_PER_TILE)])


def _edge_mlp_body(e_ref, s_ref, r_ref, we_e, we_s, we_r, be1, gg, gb,
                   we2, be2, out_ref):
    bf = jnp.bfloat16
    x = (jnp.dot(e_ref[...].astype(bf), we_e[...].astype(bf),
                 preferred_element_type=jnp.float32)
         + jnp.dot(s_ref[...].astype(bf), we_s[...].astype(bf),
                   preferred_element_type=jnp.float32)
         + jnp.dot(r_ref[...].astype(bf), we_r[...].astype(bf),
                   preferred_element_type=jnp.float32)
         + be1[...])
    m = jnp.mean(x, axis=-1, keepdims=True)
    v = jnp.mean((x - m) * (x - m), axis=-1, keepdims=True)
    h = (x - m) * lax.rsqrt(v + _EPS) * gg[...] + gb[...]
    h = jnp.maximum(h, 0.0)
    out_ref[...] = (jnp.dot(h.astype(bf), we2[...].astype(bf),
                            preferred_element_type=jnp.float32) + be2[...])


_EBLK = 4000


def _edge_mlp(edges_ac, s_feat, r_feat, we_e, we_s, we_r, be1, gg, gb,
              we2, be2):
    grid = (N_EDGES // _EBLK,)
    blk = lambda i: (i, 0)
    full = lambda i: (0, 0)
    return pl.pallas_call(
        _edge_mlp_body,
        grid=grid,
        in_specs=[
            pl.BlockSpec((_EBLK, ED), blk),
            pl.BlockSpec((_EBLK, ND), blk),
            pl.BlockSpec((_EBLK, ND), blk),
            pl.BlockSpec((ED, HID), full),
            pl.BlockSpec((ND, HID), full),
            pl.BlockSpec((ND, HID), full),
            pl.BlockSpec((1, HID), full),
            pl.BlockSpec((1, HID), full),
            pl.BlockSpec((1, HID), full),
            pl.BlockSpec((HID, ED), full),
            pl.BlockSpec((1, ED), full),
        ],
        out_specs=pl.BlockSpec((_EBLK, ED), blk),
        out_shape=jax.ShapeDtypeStruct((N_EDGES, ED), jnp.float32),
    )(edges_ac, s_feat, r_feat, we_e, we_s, we_r, be1, gg, gb, we2, be2)


def _node_mlp_body(n_ref, a0_ref, a1_ref, wn_n, wn_a, bn1, gg, gb,
                   wn2, bn2, out_ref):
    bf = jnp.bfloat16
    agg = a0_ref[...] + a1_ref[...]
    x = (jnp.dot(n_ref[...].astype(bf), wn_n[...].astype(bf),
                 preferred_element_type=jnp.float32)
         + jnp.dot(agg.astype(bf), wn_a[...].astype(bf),
                   preferred_element_type=jnp.float32)
         + bn1[...])
    m = jnp.mean(x, axis=-1, keepdims=True)
    v = jnp.mean((x - m) * (x - m), axis=-1, keepdims=True)
    h = (x - m) * lax.rsqrt(v + _EPS) * gg[...] + gb[...]
    h = jnp.maximum(h, 0.0)
    out_ref[...] = (jnp.dot(h.astype(bf), wn2[...].astype(bf),
                            preferred_element_type=jnp.float32) + bn2[...])


_NBLK = 1000


def _node_mlp(nodes, agg0, agg1, wn_n, wn_a, bn1, gg, gb, wn2, bn2):
    grid = (N_NODES // _NBLK,)
    blk = lambda i: (i, 0)
    full = lambda i: (0, 0)
    return pl.pallas_call(
        _node_mlp_body,
        grid=grid,
        in_specs=[
            pl.BlockSpec((_NBLK, ND), blk),
            pl.BlockSpec((_NBLK, ED), blk),
            pl.BlockSpec((_NBLK, ED), blk),
            pl.BlockSpec((ND, HID), full),
            pl.BlockSpec((ED, HID), full),
            pl.BlockSpec((1, HID), full),
            pl.BlockSpec((1, HID), full),
            pl.BlockSpec((1, HID), full),
            pl.BlockSpec((HID, ND), full),
            pl.BlockSpec((1, ND), full),
        ],
        out_specs=pl.BlockSpec((_NBLK, ND), blk),
        out_shape=jax.ShapeDtypeStruct((N_NODES, ND), jnp.float32),
    )(nodes, agg0, agg1, wn_n, wn_a, bn1, gg, gb, wn2, bn2)


def kernel(nodes_bus, edges_ac, edge_index, We1, be1, ge_g, ge_b, We2, be2,
           Wn1, bn1, gn_g, gn_b, Wn2, bn2):
    senders = edge_index[0]
    receivers = edge_index[1]

    s_feat, r_feat = _gather_sc(senders, receivers, nodes_bus)

    updated_edges = _edge_mlp(
        edges_ac, s_feat, r_feat,
        We1[:ED], We1[ED:ED + ND], We1[ED + ND:],
        be1.reshape(1, HID), ge_g.reshape(1, HID), ge_b.reshape(1, HID),
        We2, be2.reshape(1, ED))

    zeros = jnp.zeros((N_PAD, ED), jnp.float32)
    agg = _scatter_sc(receivers, updated_edges, zeros)

    updated_nodes = _node_mlp(
        nodes_bus, agg[0], agg[1],
        Wn1[:ND], Wn1[ND:],
        bn1.reshape(1, HID), gn_g.reshape(1, HID), gn_b.reshape(1, HID),
        Wn2, bn2.reshape(1, ND))

    return (updated_nodes, updated_edges)


# speedup vs baseline: 1.3459x; 1.0122x over previous
import functools

import jax
import jax.numpy as jnp
from jax import lax
from jax.experimental import pallas as pl
from jax.experimental.pallas import tpu as pltpu
from jax.experimental.pallas import tpu_sc as plsc

N_NODES = 10000
N_EDGES = 320000
ND = 128
ED = 128
HID = 256

NC, NS = 2, 16
NW = NC * NS
E_PER_W = N_EDGES // NW
CHUNK = 80
N_CHUNKS = E_PER_W // CHUNK
N_PAD = 10240
ROWS_PER_TILE = N_PAD // NS
CHUNK_SC = 40
N_CHUNKS_SC = E_PER_W // CHUNK_SC

_MESH = plsc.VectorSubcoreMesh(
    core_axis_name="c", subcore_axis_name="s", num_cores=NC, num_subcores=NS)

_EPS = 1e-5


NBUF = 5
CHUNK_G = 40
N_CHUNKS_G = E_PER_W // CHUNK_G
STAGE_ROWS = 640
LAST_STAGE = N_NODES - 15 * STAGE_ROWS


@functools.partial(
    pl.kernel,
    out_type=[
        jax.ShapeDtypeStruct((N_EDGES, ND), jnp.float32),
        jax.ShapeDtypeStruct((N_EDGES, ND), jnp.float32),
    ],
    mesh=_MESH,
    scratch_types=[
        pltpu.VMEM((E_PER_W,), jnp.int32),
        pltpu.VMEM((NBUF, CHUNK_G, ND), jnp.float32),
        pltpu.VMEM_SHARED((N_NODES, ND), jnp.float32),
        pltpu.SemaphoreType.DMA((NBUF,)),
        pltpu.SemaphoreType.DMA((NBUF,)),
    ],
)
def _gather_sc(senders, receivers, nodes, s_out, r_out,
               idx, buf, table, gsem, wsem):
    cid = lax.axis_index("c")
    sid = lax.axis_index("s")
    wid = sid * NC + cid
    base = wid * E_PER_W

    @pl.when(sid < NS - 1)
    def _():
        pltpu.sync_copy(nodes.at[pl.ds(sid * STAGE_ROWS, STAGE_ROWS)],
                        table.at[pl.ds(sid * STAGE_ROWS, STAGE_ROWS)])

    @pl.when(sid == NS - 1)
    def _():
        pltpu.sync_copy(nodes.at[pl.ds(15 * STAGE_ROWS, LAST_STAGE)],
                        table.at[pl.ds(15 * STAGE_ROWS, LAST_STAGE)])

    plsc.subcore_barrier()

    def one_pass(idx_hbm, out_hbm):
        pltpu.sync_copy(idx_hbm.at[pl.ds(base, E_PER_W)], idx)

        def fire_gather(c, b):
            pltpu.async_copy(table.at[idx.at[pl.ds(c * CHUNK_G, CHUNK_G)]],
                             buf.at[b], gsem.at[b])

        for b in range(NBUF):
            fire_gather(b, b)

        @pl.loop(NBUF, N_CHUNKS_G + NBUF, step=NBUF)
        def _(i):
            for b in range(NBUF):
                c = i - NBUF + b
                off = base + c * CHUNK_G
                pltpu.make_async_copy(
                    table.at[idx.at[pl.ds(c * CHUNK_G, CHUNK_G)]],
                    buf.at[b], gsem.at[b]).wait()
                pltpu.async_copy(buf.at[b], out_hbm.at[pl.ds(off, CHUNK_G)],
                                 wsem.at[b])
            for b in range(NBUF):
                c = i - NBUF + b
                off = base + c * CHUNK_G
                pltpu.make_async_copy(buf.at[b],
                                      out_hbm.at[pl.ds(off, CHUNK_G)],
                                      wsem.at[b]).wait()

                @pl.when(i + b < N_CHUNKS_G)
                def _():
                    fire_gather(i + b, b)

    one_pass(senders, s_out)
    one_pass(receivers, r_out)


@functools.partial(
    pl.kernel,
    out_type=jax.ShapeDtypeStruct((NC, N_PAD, ED), jnp.float32),
    mesh=_MESH,
    scratch_types=[
        pltpu.VMEM((NBUF, CHUNK_SC), jnp.int32),
        pltpu.VMEM((NBUF, CHUNK_SC, ED), jnp.float32),
        pltpu.VMEM_SHARED((N_PAD, ED), jnp.float32),
        pltpu.SemaphoreType.DMA((NBUF,)),
        pltpu.SemaphoreType.DMA((NBUF,)),
    ],
)
def _scatter_sc(receivers, msgs, zeros, agg_out, ibuf, mbuf, acc, isem, msem):
    cid = lax.axis_index("c")
    sid = lax.axis_index("s")
    wid = sid * NC + cid
    stripe = sid * ROWS_PER_TILE
    pltpu.sync_copy(zeros.at[pl.ds(stripe, ROWS_PER_TILE)],
                    acc.at[pl.ds(stripe, ROWS_PER_TILE)])
    plsc.subcore_barrier()

    base = wid * E_PER_W

    def fire_load(c, b):
        off = base + c * CHUNK_SC
        pltpu.async_copy(receivers.at[pl.ds(off, CHUNK_SC)], ibuf.at[b],
                         isem.at[b])
        pltpu.async_copy(msgs.at[pl.ds(off, CHUNK_SC)], mbuf.at[b],
                         msem.at[b])

    for b in range(NBUF):
        fire_load(b, b)

    @pl.loop(NBUF, N_CHUNKS_SC + NBUF, step=NBUF)
    def _(i):
        for b in range(NBUF):
            c = i - NBUF + b
            off = base + c * CHUNK_SC
            pltpu.make_async_copy(receivers.at[pl.ds(off, CHUNK_SC)],
                                  ibuf.at[b], isem.at[b]).wait()
            pltpu.make_async_copy(msgs.at[pl.ds(off, CHUNK_SC)],
                                  mbuf.at[b], msem.at[b]).wait()
            pltpu.sync_copy(mbuf.at[b], acc.at[ibuf.at[b]], add=True)

            @pl.when(i + b < N_CHUNKS_SC)
            def _():
                fire_load(i + b, b)

    plsc.subcore_barrier()
    pltpu.sync_copy(acc.at[pl.ds(stripe, ROWS_PER_TILE)],
                    agg_out.at[cid, pl.ds(stripe, ROWS_PER_TILE)])


def _edge_mlp_body(e_ref, s_ref, r_ref, we_e, we_s, we_r, be1, gg, gb,
                   we2, be2, out_ref):
    bf = jnp.bfloat16
    x = (jnp.dot(e_ref[...].astype(bf), we_e[...].astype(bf),
                 preferred_element_type=jnp.float32)
         + jnp.dot(s_ref[...].astype(bf), we_s[...].astype(bf),
                   preferred_element_type=jnp.float32)
         + jnp.dot(r_ref[...].astype(bf), we_r[...].astype(bf),
                   preferred_element_type=jnp.float32)
         + be1[...])
    m = jnp.mean(x, axis=-1, keepdims=True)
    v = jnp.mean((x - m) * (x - m), axis=-1, keepdims=True)
    h = (x - m) * lax.rsqrt(v + _EPS) * gg[...] + gb[...]
    h = jnp.maximum(h, 0.0)
    out_ref[...] = (jnp.dot(h.astype(bf), we2[...].astype(bf),
                            preferred_element_type=jnp.float32) + be2[...])


_EBLK = 4000


def _edge_mlp(edges_ac, s_feat, r_feat, we_e, we_s, we_r, be1, gg, gb,
              we2, be2):
    grid = (N_EDGES // _EBLK,)
    blk = lambda i: (i, 0)
    full = lambda i: (0, 0)
    return pl.pallas_call(
        _edge_mlp_body,
        grid=grid,
        in_specs=[
            pl.BlockSpec((_EBLK, ED), blk),
            pl.BlockSpec((_EBLK, ND), blk),
            pl.BlockSpec((_EBLK, ND), blk),
            pl.BlockSpec((ED, HID), full),
            pl.BlockSpec((ND, HID), full),
            pl.BlockSpec((ND, HID), full),
            pl.BlockSpec((1, HID), full),
            pl.BlockSpec((1, HID), full),
            pl.BlockSpec((1, HID), full),
            pl.BlockSpec((HID, ED), full),
            pl.BlockSpec((1, ED), full),
        ],
        out_specs=pl.BlockSpec((_EBLK, ED), blk),
        out_shape=jax.ShapeDtypeStruct((N_EDGES, ED), jnp.float32),
    )(edges_ac, s_feat, r_feat, we_e, we_s, we_r, be1, gg, gb, we2, be2)


def _node_mlp_body(n_ref, a0_ref, a1_ref, wn_n, wn_a, bn1, gg, gb,
                   wn2, bn2, out_ref):
    bf = jnp.bfloat16
    agg = a0_ref[0] + a1_ref[0]
    x = (jnp.dot(n_ref[...].astype(bf), wn_n[...].astype(bf),
                 preferred_element_type=jnp.float32)
         + jnp.dot(agg.astype(bf), wn_a[...].astype(bf),
                   preferred_element_type=jnp.float32)
         + bn1[...])
    m = jnp.mean(x, axis=-1, keepdims=True)
    v = jnp.mean((x - m) * (x - m), axis=-1, keepdims=True)
    h = (x - m) * lax.rsqrt(v + _EPS) * gg[...] + gb[...]
    h = jnp.maximum(h, 0.0)
    out_ref[...] = (jnp.dot(h.astype(bf), wn2[...].astype(bf),
                            preferred_element_type=jnp.float32) + bn2[...])


_NBLK = 1000


def _node_mlp(nodes, agg, wn_n, wn_a, bn1, gg, gb, wn2, bn2):
    grid = (N_NODES // _NBLK,)
    blk = lambda i: (i, 0)
    full = lambda i: (0, 0)
    return pl.pallas_call(
        _node_mlp_body,
        grid=grid,
        in_specs=[
            pl.BlockSpec((_NBLK, ND), blk),
            pl.BlockSpec((1, _NBLK, ED), lambda i: (0, i, 0)),
            pl.BlockSpec((1, _NBLK, ED), lambda i: (1, i, 0)),
            pl.BlockSpec((ND, HID), full),
            pl.BlockSpec((ED, HID), full),
            pl.BlockSpec((1, HID), full),
            pl.BlockSpec((1, HID), full),
            pl.BlockSpec((1, HID), full),
            pl.BlockSpec((HID, ND), full),
            pl.BlockSpec((1, ND), full),
        ],
        out_specs=pl.BlockSpec((_NBLK, ND), blk),
        out_shape=jax.ShapeDtypeStruct((N_NODES, ND), jnp.float32),
    )(nodes, agg, agg, wn_n, wn_a, bn1, gg, gb, wn2, bn2)


def kernel(nodes_bus, edges_ac, edge_index, We1, be1, ge_g, ge_b, We2, be2,
           Wn1, bn1, gn_g, gn_b, Wn2, bn2):
    senders = edge_index[0]
    receivers = edge_index[1]

    s_feat, r_feat = _gather_sc(senders, receivers, nodes_bus)

    updated_edges = _edge_mlp(
        edges_ac, s_feat, r_feat,
        We1[:ED], We1[ED:ED + ND], We1[ED + ND:],
        be1.reshape(1, HID), ge_g.reshape(1, HID), ge_b.reshape(1, HID),
        We2, be2.reshape(1, ED))

    zeros = jnp.zeros((N_PAD, ED), jnp.float32)
    agg = _scatter_sc(receivers, updated_edges, zeros)

    updated_nodes = _node_mlp(
        nodes_bus, agg,
        Wn1[:ND], Wn1[ND:],
        bn1.reshape(1, HID), gn_g.reshape(1, HID), gn_b.reshape(1, HID),
        Wn2, bn2.reshape(1, ND))

    return (updated_nodes, updated_edges)


# speedup vs baseline: 1.4066x; 1.0451x over previous
import functools

import jax
import jax.numpy as jnp
from jax import lax
from jax.experimental import pallas as pl
from jax.experimental.pallas import tpu as pltpu
from jax.experimental.pallas import tpu_sc as plsc

N_NODES = 10000
N_EDGES = 320000
ND = 128
ED = 128
HID = 256

NC, NS = 2, 16
NW = NC * NS
E_PER_W = N_EDGES // NW
CHUNK = 80
N_CHUNKS = E_PER_W // CHUNK
N_PAD = 10240
ROWS_PER_TILE = N_PAD // NS
CHUNK_SC = 40
N_CHUNKS_SC = E_PER_W // CHUNK_SC

_MESH = plsc.VectorSubcoreMesh(
    core_axis_name="c", subcore_axis_name="s", num_cores=NC, num_subcores=NS)

_EPS = 1e-5


NBUF = 5
CHUNK_G = 40
N_CHUNKS_G = E_PER_W // CHUNK_G
STAGE_ROWS = 640
LAST_STAGE = N_NODES - 15 * STAGE_ROWS


@functools.partial(
    pl.kernel,
    out_type=[
        jax.ShapeDtypeStruct((N_EDGES, ND), jnp.float32),
        jax.ShapeDtypeStruct((N_EDGES, ND), jnp.float32),
    ],
    mesh=_MESH,
    scratch_types=[
        pltpu.VMEM((E_PER_W,), jnp.int32),
        pltpu.VMEM((NBUF, CHUNK_G, ND), jnp.float32),
        pltpu.VMEM_SHARED((N_NODES, ND), jnp.float32),
        pltpu.SemaphoreType.DMA((NBUF,)),
        pltpu.SemaphoreType.DMA((NBUF,)),
    ],
)
def _gather_sc(senders, receivers, nodes, s_out, r_out,
               idx, buf, table, gsem, wsem):
    cid = lax.axis_index("c")
    sid = lax.axis_index("s")
    wid = sid * NC + cid
    base = wid * E_PER_W

    @pl.when(sid < NS - 1)
    def _():
        pltpu.sync_copy(nodes.at[pl.ds(sid * STAGE_ROWS, STAGE_ROWS)],
                        table.at[pl.ds(sid * STAGE_ROWS, STAGE_ROWS)])

    @pl.when(sid == NS - 1)
    def _():
        pltpu.sync_copy(nodes.at[pl.ds(15 * STAGE_ROWS, LAST_STAGE)],
                        table.at[pl.ds(15 * STAGE_ROWS, LAST_STAGE)])

    plsc.subcore_barrier()

    def one_pass(idx_hbm, out_hbm):
        pltpu.sync_copy(idx_hbm.at[pl.ds(base, E_PER_W)], idx)

        def fire_gather(c, b):
            pltpu.async_copy(table.at[idx.at[pl.ds(c * CHUNK_G, CHUNK_G)]],
                             buf.at[b], gsem.at[b])

        for b in range(NBUF):
            fire_gather(b, b)

        @pl.loop(NBUF, N_CHUNKS_G + NBUF, step=NBUF)
        def _(i):
            for b in range(NBUF):
                c = i - NBUF + b
                off = base + c * CHUNK_G
                pltpu.make_async_copy(
                    table.at[idx.at[pl.ds(c * CHUNK_G, CHUNK_G)]],
                    buf.at[b], gsem.at[b]).wait()
                pltpu.async_copy(buf.at[b], out_hbm.at[pl.ds(off, CHUNK_G)],
                                 wsem.at[b])
            for b in range(NBUF):
                c = i - NBUF + b
                off = base + c * CHUNK_G
                pltpu.make_async_copy(buf.at[b],
                                      out_hbm.at[pl.ds(off, CHUNK_G)],
                                      wsem.at[b]).wait()

                @pl.when(i + b < N_CHUNKS_G)
                def _():
                    fire_gather(i + b, b)

    one_pass(senders, s_out)
    one_pass(receivers, r_out)


@functools.partial(
    pl.kernel,
    out_type=jax.ShapeDtypeStruct((NC, N_PAD, ED), jnp.float32),
    mesh=_MESH,
    scratch_types=[
        pltpu.VMEM((NBUF, CHUNK_SC), jnp.int32),
        pltpu.VMEM((NBUF, CHUNK_SC, ED), jnp.float32),
        pltpu.VMEM_SHARED((N_PAD, ED), jnp.float32),
        pltpu.SemaphoreType.DMA((NBUF,)),
        pltpu.SemaphoreType.DMA((NBUF,)),
    ],
)
def _scatter_sc(receivers, msgs, zeros, agg_out, ibuf, mbuf, acc, isem, msem):
    cid = lax.axis_index("c")
    sid = lax.axis_index("s")
    wid = sid * NC + cid
    stripe = sid * ROWS_PER_TILE
    pltpu.sync_copy(zeros.at[pl.ds(stripe, ROWS_PER_TILE)],
                    acc.at[pl.ds(stripe, ROWS_PER_TILE)])
    plsc.subcore_barrier()

    base = wid * E_PER_W

    def fire_load(c, b):
        off = base + c * CHUNK_SC
        pltpu.async_copy(receivers.at[pl.ds(off, CHUNK_SC)], ibuf.at[b],
                         isem.at[b])
        pltpu.async_copy(msgs.at[pl.ds(off, CHUNK_SC)], mbuf.at[b],
                         msem.at[b])

    for b in range(NBUF):
        fire_load(b, b)

    @pl.loop(NBUF, N_CHUNKS_SC + NBUF, step=NBUF)
    def _(i):
        for b in range(NBUF):
            c = i - NBUF + b
            off = base + c * CHUNK_SC
            pltpu.make_async_copy(receivers.at[pl.ds(off, CHUNK_SC)],
                                  ibuf.at[b], isem.at[b]).wait()
            pltpu.make_async_copy(msgs.at[pl.ds(off, CHUNK_SC)],
                                  mbuf.at[b], msem.at[b]).wait()
            pltpu.sync_copy(mbuf.at[b], acc.at[ibuf.at[b]], add=True)

            @pl.when(i + b < N_CHUNKS_SC)
            def _():
                fire_load(i + b, b)

    plsc.subcore_barrier()
    pltpu.sync_copy(acc.at[pl.ds(stripe, ROWS_PER_TILE)],
                    agg_out.at[cid, pl.ds(stripe, ROWS_PER_TILE)])


def _edge_mlp_body(e_ref, s_ref, r_ref, we_e, we_s, we_r, be1, gg, gb,
                   we2, be2, out_ref):
    bf = jnp.bfloat16
    x = (jnp.dot(e_ref[...].astype(bf), we_e[...].astype(bf),
                 preferred_element_type=jnp.float32)
         + jnp.dot(s_ref[...].astype(bf), we_s[...].astype(bf),
                   preferred_element_type=jnp.float32)
         + jnp.dot(r_ref[...].astype(bf), we_r[...].astype(bf),
                   preferred_element_type=jnp.float32)
         + be1[...])
    m = jnp.mean(x, axis=-1, keepdims=True)
    v = jnp.mean((x - m) * (x - m), axis=-1, keepdims=True)
    h = (x - m) * lax.rsqrt(v + _EPS) * gg[...] + gb[...]
    h = jnp.maximum(h, 0.0)
    out_ref[...] = (jnp.dot(h.astype(bf), we2[...].astype(bf),
                            preferred_element_type=jnp.float32) + be2[...])


_EBLK = 8000


def _edge_mlp(edges_ac, s_feat, r_feat, we_e, we_s, we_r, be1, gg, gb,
              we2, be2):
    grid = (N_EDGES // _EBLK,)
    blk = lambda i: (i, 0)
    full = lambda i: (0, 0)
    return pl.pallas_call(
        _edge_mlp_body,
        grid=grid,
        in_specs=[
            pl.BlockSpec((_EBLK, ED), blk),
            pl.BlockSpec((_EBLK, ND), blk),
            pl.BlockSpec((_EBLK, ND), blk),
            pl.BlockSpec((ED, HID), full),
            pl.BlockSpec((ND, HID), full),
            pl.BlockSpec((ND, HID), full),
            pl.BlockSpec((1, HID), full),
            pl.BlockSpec((1, HID), full),
            pl.BlockSpec((1, HID), full),
            pl.BlockSpec((HID, ED), full),
            pl.BlockSpec((1, ED), full),
        ],
        out_specs=pl.BlockSpec((_EBLK, ED), blk),
        out_shape=jax.ShapeDtypeStruct((N_EDGES, ED), jnp.float32),
    )(edges_ac, s_feat, r_feat, we_e, we_s, we_r, be1, gg, gb, we2, be2)


def _node_mlp_body(n_ref, a0_ref, a1_ref, wn_n, wn_a, bn1, gg, gb,
                   wn2, bn2, out_ref):
    bf = jnp.bfloat16
    agg = a0_ref[0] + a1_ref[0]
    x = (jnp.dot(n_ref[...].astype(bf), wn_n[...].astype(bf),
                 preferred_element_type=jnp.float32)
         + jnp.dot(agg.astype(bf), wn_a[...].astype(bf),
                   preferred_element_type=jnp.float32)
         + bn1[...])
    m = jnp.mean(x, axis=-1, keepdims=True)
    v = jnp.mean((x - m) * (x - m), axis=-1, keepdims=True)
    h = (x - m) * lax.rsqrt(v + _EPS) * gg[...] + gb[...]
    h = jnp.maximum(h, 0.0)
    out_ref[...] = (jnp.dot(h.astype(bf), wn2[...].astype(bf),
                            preferred_element_type=jnp.float32) + bn2[...])


_NBLK = 1000


def _node_mlp(nodes, agg, wn_n, wn_a, bn1, gg, gb, wn2, bn2):
    grid = (N_NODES // _NBLK,)
    blk = lambda i: (i, 0)
    full = lambda i: (0, 0)
    return pl.pallas_call(
        _node_mlp_body,
        grid=grid,
        in_specs=[
            pl.BlockSpec((_NBLK, ND), blk),
            pl.BlockSpec((1, _NBLK, ED), lambda i: (0, i, 0)),
            pl.BlockSpec((1, _NBLK, ED), lambda i: (1, i, 0)),
            pl.BlockSpec((ND, HID), full),
            pl.BlockSpec((ED, HID), full),
            pl.BlockSpec((1, HID), full),
            pl.BlockSpec((1, HID), full),
            pl.BlockSpec((1, HID), full),
            pl.BlockSpec((HID, ND), full),
            pl.BlockSpec((1, ND), full),
        ],
        out_specs=pl.BlockSpec((_NBLK, ND), blk),
        out_shape=jax.ShapeDtypeStruct((N_NODES, ND), jnp.float32),
    )(nodes, agg, agg, wn_n, wn_a, bn1, gg, gb, wn2, bn2)


def kernel(nodes_bus, edges_ac, edge_index, We1, be1, ge_g, ge_b, We2, be2,
           Wn1, bn1, gn_g, gn_b, Wn2, bn2):
    senders = edge_index[0]
    receivers = edge_index[1]

    s_feat, r_feat = _gather_sc(senders, receivers, nodes_bus)

    updated_edges = _edge_mlp(
        edges_ac, s_feat, r_feat,
        We1[:ED], We1[ED:ED + ND], We1[ED + ND:],
        be1.reshape(1, HID), ge_g.reshape(1, HID), ge_b.reshape(1, HID),
        We2, be2.reshape(1, ED))

    zeros = jnp.zeros((N_PAD, ED), jnp.float32)
    agg = _scatter_sc(receivers, updated_edges, zeros)

    updated_nodes = _node_mlp(
        nodes_bus, agg,
        Wn1[:ND], Wn1[ND:],
        bn1.reshape(1, HID), gn_g.reshape(1, HID), gn_b.reshape(1, HID),
        Wn2, bn2.reshape(1, ND))

    return (updated_nodes, updated_edges)
